# baseline (speedup 1.0000x reference)
import functools

import jax
import jax.numpy as jnp
from jax import lax
from jax.experimental import pallas as pl
from jax.experimental.pallas import tpu as pltpu
from jax.experimental.pallas import tpu_sc as plsc

EMB = 64
NC = 2
NS = 16
NW = NC * NS
BL = 128
KL = 2
CH = KL * BL
UNROLL = 4


def _transpose_block(rows_v, t_v, lc, row_base):
    iota16 = lax.iota(jnp.int32, 16)

    def b16_body(b16, carry):
        ridx = row_base + b16 * 16 + iota16
        boff = b16 * 16

        def e_body(e, carry2):
            cidx = jnp.full((16,), e, jnp.int32)
            v = plsc.load_gather(rows_v, [ridx, cidx])
            t_v[lc, e, pl.ds(boff, 16)] = v
            return carry2

        return lax.fori_loop(0, EMB, e_body, carry, unroll=16)

    lax.fori_loop(0, BL // 16, b16_body, 0)


def _make_gather(n_user: int, b: int, l: int):
    assert b % (NW * 16) == 0
    bpw = b // NW
    nblk = bpw // BL
    assert l % KL == 0
    nlch = l // KL
    n_chunks = nblk * nlch
    assert n_chunks % UNROLL == 0
    assert n_user == b

    mesh = plsc.VectorSubcoreMesh(core_axis_name="c", subcore_axis_name="s")

    @functools.partial(
        pl.kernel,
        mesh=mesh,
        compiler_params=pltpu.CompilerParams(
            use_tc_tiling_on_sc=False, needs_layout_passes=False),
        out_type=[
            jax.ShapeDtypeStruct((EMB, b), jnp.float32),
            jax.ShapeDtypeStruct((l, EMB, b), jnp.float32),
        ],
        scratch_types=[
            pltpu.VMEM((CH,), jnp.int32),
            pltpu.VMEM((CH,), jnp.int32),
            pltpu.VMEM((CH,), jnp.int32),
            pltpu.VMEM((CH,), jnp.int32),
            pltpu.VMEM((CH, EMB), jnp.float32),
            pltpu.VMEM((CH, EMB), jnp.float32),
            pltpu.VMEM((KL, EMB, BL), jnp.float32),
            pltpu.VMEM((KL, EMB, BL), jnp.float32),
            pltpu.SemaphoreType.DMA,
            pltpu.SemaphoreType.DMA,
            pltpu.SemaphoreType.DMA,
            pltpu.SemaphoreType.DMA,
            pltpu.SemaphoreType.DMA,
            pltpu.SemaphoreType.DMA,
            pltpu.SemaphoreType.DMA,
            pltpu.SemaphoreType.DMA,
        ],
    )
    def gather(uids, iidsT, utab, itab, uout, iout,
               i0, i1, i2, i3, r0, r1, t0, t1,
               si0, si1, si2, si3, sg0, sg1, so0, so1):
        idx = (i0, i1, i2, i3)
        rows = (r0, r1)
        tb = (t0, t1)
        si = (si0, si1, si2, si3)
        sg = (sg0, sg1)
        so = (so0, so1)

        wid = lax.axis_index("s") * NC + lax.axis_index("c")
        bbase = wid * bpw

        def user_body(k, carry):
            ub0 = pl.multiple_of(bbase + k * BL, 8)
            pltpu.sync_copy(uids.at[pl.ds(ub0, BL)], i0.at[pl.ds(0, BL)])
            pltpu.async_copy(
                utab.at[i0.at[pl.ds(0, BL)]], r0.at[pl.ds(0, BL)], sg0
            ).wait()
            _transpose_block(r0, t0, 0, 0)
            pltpu.sync_copy(t0.at[0], uout.at[:, pl.ds(ub0, BL)])
            return carry

        lax.fori_loop(0, nblk, user_body, 0)

        def idx_starts(c, q):
            bi = lax.rem(c, nblk)
            l0 = lax.div(c, nblk) * KL
            b0 = bbase + bi * BL
            for lc in range(KL):
                pltpu.make_async_copy(
                    iidsT.at[l0 + lc, pl.ds(b0, BL)],
                    idx[q].at[pl.ds(lc * BL, BL)], si[q]).start()

        def idx_waits(c, q):
            bi = lax.rem(c, nblk)
            l0 = lax.div(c, nblk) * KL
            b0 = bbase + bi * BL
            for lc in range(KL):
                pltpu.make_async_copy(
                    iidsT.at[l0 + lc, pl.ds(b0, BL)],
                    idx[q].at[pl.ds(lc * BL, BL)], si[q]).wait()

        def store_op(c, p):
            bi = lax.rem(c, nblk)
            l0 = lax.div(c, nblk) * KL
            b0 = bbase + bi * BL
            return pltpu.make_async_copy(
                tb[p], iout.at[pl.ds(l0, KL), :, pl.ds(b0, BL)], so[p])

        for q in range(UNROLL):
            idx_starts(q, q)

        def outer(go, carry):
            for j in range(UNROLL):
                c = go * UNROLL + j
                @pl.when(c < n_chunks)
                def _():
                    idx_waits(c, j)
                    pltpu.make_async_copy(
                        itab.at[idx[j]], rows[j % 2], sg[j % 2]).start()
                cf = c - 1
                pj = (j + 1) % 2
                qf = (j + 3) % 4

                @pl.when((cf >= 0) & (cf < n_chunks))
                def _():
                    pltpu.make_async_copy(
                        itab.at[idx[qf]], rows[pj], sg[pj]).wait()
                    @pl.when(cf + UNROLL < n_chunks)
                    def _():
                        idx_starts(cf + UNROLL, qf)
                    @pl.when(cf >= 2)
                    def _():
                        store_op(cf - 2, pj).wait()
                    for lc in range(KL):
                        _transpose_block(rows[pj], tb[pj], lc, lc * BL)
                    store_op(cf, pj).start()
            return carry

        lax.fori_loop(0, n_chunks // UNROLL + 1, outer, 0)

        store_op(n_chunks - 2, (n_chunks - 2) % 2).wait()
        store_op(n_chunks - 1, (n_chunks - 1) % 2).wait()

    return gather


def kernel(user_ids, item_ids, user_table, item_table):
    b, l = item_ids.shape
    iT = jnp.transpose(item_ids).astype(jnp.int32)
    uids = user_ids.astype(jnp.int32)
    gather = _make_gather(uids.shape[0], b, l)
    uout_t, iout_t = gather(uids, iT, user_table, item_table)
    return (jnp.transpose(uout_t), jnp.transpose(iout_t, (2, 0, 1)))


# speedup vs baseline: 1.9201x; 1.9201x over previous
import functools

import jax
import jax.numpy as jnp
from jax import lax
from jax.experimental import pallas as pl
from jax.experimental.pallas import tpu as pltpu
from jax.experimental.pallas import tpu_sc as plsc

EMB = 64
NC = 2
NS = 16
NW = NC * NS
BL = 128
BLP = 129
KL = 2
CH = KL * BL
UNROLL = 4


def _transpose_block(rows_v, t_v, lc, row_base):
    iota16 = lax.iota(jnp.int32, 16)
    lcv = jnp.full((16,), lc, jnp.int32)

    def b_body(b, carry):
        bv = jnp.full((16,), b, jnp.int32)
        r = row_base + b
        for e0 in range(EMB // 16):
            v = rows_v[r, pl.ds(e0 * 16, 16)]
            plsc.store_scatter(t_v, [lcv, e0 * 16 + iota16, bv], v)
        return carry

    lax.fori_loop(0, BL, b_body, 0, unroll=8)


def _make_gather(n_user: int, b: int, l: int):
    assert b % (NW * 16) == 0
    bpw = b // NW
    nblk = bpw // BL
    assert l % KL == 0
    nlch = l // KL
    n_chunks = nblk * nlch
    assert n_chunks % UNROLL == 0
    assert n_user == b

    mesh = plsc.VectorSubcoreMesh(core_axis_name="c", subcore_axis_name="s")

    @functools.partial(
        pl.kernel,
        mesh=mesh,
        compiler_params=pltpu.CompilerParams(
            use_tc_tiling_on_sc=False, needs_layout_passes=False),
        out_type=[
            jax.ShapeDtypeStruct((EMB, b), jnp.float32),
            jax.ShapeDtypeStruct((l, EMB, b), jnp.float32),
        ],
        scratch_types=[
            pltpu.VMEM((CH,), jnp.int32),
            pltpu.VMEM((CH,), jnp.int32),
            pltpu.VMEM((CH,), jnp.int32),
            pltpu.VMEM((CH,), jnp.int32),
            pltpu.VMEM((CH, EMB), jnp.float32),
            pltpu.VMEM((CH, EMB), jnp.float32),
            pltpu.VMEM((KL, EMB, BLP), jnp.float32),
            pltpu.VMEM((KL, EMB, BLP), jnp.float32),
            pltpu.SemaphoreType.DMA,
            pltpu.SemaphoreType.DMA,
            pltpu.SemaphoreType.DMA,
            pltpu.SemaphoreType.DMA,
            pltpu.SemaphoreType.DMA,
            pltpu.SemaphoreType.DMA,
            pltpu.SemaphoreType.DMA,
            pltpu.SemaphoreType.DMA,
        ],
    )
    def gather(uids, iidsT, utab, itab, uout, iout,
               i0, i1, i2, i3, r0, r1, t0, t1,
               si0, si1, si2, si3, sg0, sg1, so0, so1):
        idx = (i0, i1, i2, i3)
        rows = (r0, r1)
        tb = (t0, t1)
        si = (si0, si1, si2, si3)
        sg = (sg0, sg1)
        so = (so0, so1)

        wid = lax.axis_index("s") * NC + lax.axis_index("c")
        bbase = wid * bpw

        def user_body(k, carry):
            ub0 = pl.multiple_of(bbase + k * BL, 8)
            pltpu.sync_copy(uids.at[pl.ds(ub0, BL)], i0.at[pl.ds(0, BL)])
            pltpu.async_copy(
                utab.at[i0.at[pl.ds(0, BL)]], r0.at[pl.ds(0, BL)], sg0
            ).wait()
            _transpose_block(r0, t0, 0, 0)
            pltpu.sync_copy(
                t0.at[0, :, pl.ds(0, BL)], uout.at[:, pl.ds(ub0, BL)])
            return carry

        lax.fori_loop(0, nblk, user_body, 0)

        def idx_starts(c, q):
            bi = lax.rem(c, nblk)
            l0 = lax.div(c, nblk) * KL
            b0 = bbase + bi * BL
            for lc in range(KL):
                pltpu.make_async_copy(
                    iidsT.at[l0 + lc, pl.ds(b0, BL)],
                    idx[q].at[pl.ds(lc * BL, BL)], si[q]).start()

        def idx_waits(c, q):
            bi = lax.rem(c, nblk)
            l0 = lax.div(c, nblk) * KL
            b0 = bbase + bi * BL
            for lc in range(KL):
                pltpu.make_async_copy(
                    iidsT.at[l0 + lc, pl.ds(b0, BL)],
                    idx[q].at[pl.ds(lc * BL, BL)], si[q]).wait()

        def store_op(c, p):
            bi = lax.rem(c, nblk)
            l0 = lax.div(c, nblk) * KL
            b0 = bbase + bi * BL
            return pltpu.make_async_copy(
                tb[p].at[:, :, pl.ds(0, BL)],
                iout.at[pl.ds(l0, KL), :, pl.ds(b0, BL)], so[p])

        for q in range(UNROLL):
            idx_starts(q, q)

        def outer(go, carry):
            for j in range(UNROLL):
                c = go * UNROLL + j
                @pl.when(c < n_chunks)
                def _():
                    idx_waits(c, j)
                    pltpu.make_async_copy(
                        itab.at[idx[j]], rows[j % 2], sg[j % 2]).start()
                cf = c - 1
                pj = (j + 1) % 2
                qf = (j + 3) % 4

                @pl.when((cf >= 0) & (cf < n_chunks))
                def _():
                    pltpu.make_async_copy(
                        itab.at[idx[qf]], rows[pj], sg[pj]).wait()
                    @pl.when(cf + UNROLL < n_chunks)
                    def _():
                        idx_starts(cf + UNROLL, qf)
                    @pl.when(cf >= 2)
                    def _():
                        store_op(cf - 2, pj).wait()
                    for lc in range(KL):
                        _transpose_block(rows[pj], tb[pj], lc, lc * BL)
                    store_op(cf, pj).start()
            return carry

        lax.fori_loop(0, n_chunks // UNROLL + 1, outer, 0)

        store_op(n_chunks - 2, (n_chunks - 2) % 2).wait()
        store_op(n_chunks - 1, (n_chunks - 1) % 2).wait()

    return gather


def kernel(user_ids, item_ids, user_table, item_table):
    b, l = item_ids.shape
    iT = jnp.transpose(item_ids).astype(jnp.int32)
    uids = user_ids.astype(jnp.int32)
    gather = _make_gather(uids.shape[0], b, l)
    uout_t, iout_t = gather(uids, iT, user_table, item_table)
    return (jnp.transpose(uout_t), jnp.transpose(iout_t, (2, 0, 1)))


# speedup vs baseline: 1.9262x; 1.0032x over previous
import functools

import jax
import jax.numpy as jnp
from jax import lax
from jax.experimental import pallas as pl
from jax.experimental.pallas import tpu as pltpu
from jax.experimental.pallas import tpu_sc as plsc

EMB = 64
NC = 2
NS = 16
NW = NC * NS
BL = 128
BLP = 129
KL = 2
CH = KL * BL
UNROLL = 4


def _transpose_block(rows_v, t_v, lc, row_base):
    iota16 = lax.iota(jnp.int32, 16)
    zerov = jnp.zeros((16,), jnp.int32)
    evs = [lc * EMB + e0 * 16 + iota16 for e0 in range(EMB // 16)]

    def b_body(b, carry):
        bv = jnp.full((16,), b, jnp.int32)
        r = row_base + b
        for e0 in range(EMB // 16):
            v = rows_v[r, pl.ds(e0 * 16, 16)]
            plsc.store_scatter(t_v, [zerov, evs[e0], bv], v)
        return carry

    lax.fori_loop(0, BL, b_body, 0, unroll=8)


def _make_gather(n_user: int, b: int, l: int):
    assert b % (NW * 16) == 0
    bpw = b // NW
    nblk = bpw // BL
    assert l % KL == 0
    nlch = l // KL
    n_chunks = nblk * nlch
    assert n_chunks % UNROLL == 0
    assert n_user == b

    mesh = plsc.VectorSubcoreMesh(core_axis_name="c", subcore_axis_name="s")

    @functools.partial(
        pl.kernel,
        mesh=mesh,
        compiler_params=pltpu.CompilerParams(
            use_tc_tiling_on_sc=False, needs_layout_passes=False),
        out_type=[
            jax.ShapeDtypeStruct((EMB, b), jnp.float32),
            jax.ShapeDtypeStruct((l, EMB, b), jnp.float32),
        ],
        scratch_types=[
            pltpu.VMEM((CH,), jnp.int32),
            pltpu.VMEM((CH,), jnp.int32),
            pltpu.VMEM((CH,), jnp.int32),
            pltpu.VMEM((CH,), jnp.int32),
            pltpu.VMEM((CH, EMB), jnp.float32),
            pltpu.VMEM((CH, EMB), jnp.float32),
            pltpu.VMEM((KL, EMB, BLP), jnp.float32),
            pltpu.VMEM((KL, EMB, BLP), jnp.float32),
            pltpu.SemaphoreType.DMA,
            pltpu.SemaphoreType.DMA,
            pltpu.SemaphoreType.DMA,
            pltpu.SemaphoreType.DMA,
            pltpu.SemaphoreType.DMA,
            pltpu.SemaphoreType.DMA,
            pltpu.SemaphoreType.DMA,
            pltpu.SemaphoreType.DMA,
        ],
    )
    def gather(uids, iidsT, utab, itab, uout, iout,
               i0, i1, i2, i3, r0, r1, t0, t1,
               si0, si1, si2, si3, sg0, sg1, so0, so1):
        idx = (i0, i1, i2, i3)
        rows = (r0, r1)
        tb = (t0, t1)
        si = (si0, si1, si2, si3)
        sg = (sg0, sg1)
        so = (so0, so1)

        wid = lax.axis_index("s") * NC + lax.axis_index("c")
        bbase = wid * bpw

        def user_body(k, carry):
            ub0 = pl.multiple_of(bbase + k * BL, 8)
            pltpu.sync_copy(uids.at[pl.ds(ub0, BL)], i0.at[pl.ds(0, BL)])
            pltpu.async_copy(
                utab.at[i0.at[pl.ds(0, BL)]], r0.at[pl.ds(0, BL)], sg0
            ).wait()
            _transpose_block(r0, t0, 0, 0)
            pltpu.sync_copy(
                t0.at[0, :, pl.ds(0, BL)], uout.at[:, pl.ds(ub0, BL)])
            return carry

        lax.fori_loop(0, nblk, user_body, 0)

        def idx_starts(c, q):
            bi = lax.rem(c, nblk)
            l0 = lax.div(c, nblk) * KL
            b0 = bbase + bi * BL
            for lc in range(KL):
                pltpu.make_async_copy(
                    iidsT.at[l0 + lc, pl.ds(b0, BL)],
                    idx[q].at[pl.ds(lc * BL, BL)], si[q]).start()

        def idx_waits(c, q):
            bi = lax.rem(c, nblk)
            l0 = lax.div(c, nblk) * KL
            b0 = bbase + bi * BL
            for lc in range(KL):
                pltpu.make_async_copy(
                    iidsT.at[l0 + lc, pl.ds(b0, BL)],
                    idx[q].at[pl.ds(lc * BL, BL)], si[q]).wait()

        def store_op(c, p):
            bi = lax.rem(c, nblk)
            l0 = lax.div(c, nblk) * KL
            b0 = bbase + bi * BL
            return pltpu.make_async_copy(
                tb[p].at[:, :, pl.ds(0, BL)],
                iout.at[pl.ds(l0, KL), :, pl.ds(b0, BL)], so[p])

        for q in range(UNROLL):
            idx_starts(q, q)

        def outer(go, carry):
            for j in range(UNROLL):
                c = go * UNROLL + j
                @pl.when(c < n_chunks)
                def _():
                    idx_waits(c, j)
                    pltpu.make_async_copy(
                        itab.at[idx[j]], rows[j % 2], sg[j % 2]).start()
                cf = c - 1
                pj = (j + 1) % 2
                qf = (j + 3) % 4

                @pl.when((cf >= 0) & (cf < n_chunks))
                def _():
                    pltpu.make_async_copy(
                        itab.at[idx[qf]], rows[pj], sg[pj]).wait()
                    @pl.when(cf + UNROLL < n_chunks)
                    def _():
                        idx_starts(cf + UNROLL, qf)
                    @pl.when(cf >= 2)
                    def _():
                        store_op(cf - 2, pj).wait()
                    for lc in range(KL):
                        _transpose_block(rows[pj], tb[pj], lc, lc * BL)
                    store_op(cf, pj).start()
            return carry

        lax.fori_loop(0, n_chunks // UNROLL + 1, outer, 0)

        store_op(n_chunks - 2, (n_chunks - 2) % 2).wait()
        store_op(n_chunks - 1, (n_chunks - 1) % 2).wait()

    return gather


def kernel(user_ids, item_ids, user_table, item_table):
    b, l = item_ids.shape
    iT = jnp.transpose(item_ids).astype(jnp.int32)
    uids = user_ids.astype(jnp.int32)
    gather = _make_gather(uids.shape[0], b, l)
    uout_t, iout_t = gather(uids, iT, user_table, item_table)
    return (jnp.transpose(uout_t), jnp.transpose(iout_t, (2, 0, 1)))


# speedup vs baseline: 1.9950x; 1.0357x over previous
import functools

import jax
import jax.numpy as jnp
from jax import lax
from jax.experimental import pallas as pl
from jax.experimental.pallas import tpu as pltpu
from jax.experimental.pallas import tpu_sc as plsc

EMB = 64
NC = 2
NS = 16
NW = NC * NS
BL = 256
BLP = 257
KL = 1
CH = KL * BL
UNROLL = 4


def _transpose_block(rows_v, t_v, lc, row_base):
    iota16 = lax.iota(jnp.int32, 16)
    zerov = jnp.zeros((16,), jnp.int32)
    evs = [lc * EMB + e0 * 16 + iota16 for e0 in range(EMB // 16)]

    def b_body(b, carry):
        bv = jnp.full((16,), b, jnp.int32)
        r = row_base + b
        for e0 in range(EMB // 16):
            v = rows_v[r, pl.ds(e0 * 16, 16)]
            plsc.store_scatter(t_v, [zerov, evs[e0], bv], v)
        return carry

    lax.fori_loop(0, BL, b_body, 0, unroll=8)


def _make_gather(n_user: int, b: int, l: int):
    assert b % (NW * 16) == 0
    bpw = b // NW
    nblk = bpw // BL
    assert l % KL == 0
    nlch = l // KL
    n_chunks = nblk * nlch
    assert n_chunks % UNROLL == 0
    assert n_user == b

    mesh = plsc.VectorSubcoreMesh(core_axis_name="c", subcore_axis_name="s")

    @functools.partial(
        pl.kernel,
        mesh=mesh,
        compiler_params=pltpu.CompilerParams(
            use_tc_tiling_on_sc=False, needs_layout_passes=False),
        out_type=[
            jax.ShapeDtypeStruct((EMB, b), jnp.float32),
            jax.ShapeDtypeStruct((l, EMB, b), jnp.float32),
        ],
        scratch_types=[
            pltpu.VMEM((CH,), jnp.int32),
            pltpu.VMEM((CH,), jnp.int32),
            pltpu.VMEM((CH,), jnp.int32),
            pltpu.VMEM((CH,), jnp.int32),
            pltpu.VMEM((CH, EMB), jnp.float32),
            pltpu.VMEM((CH, EMB), jnp.float32),
            pltpu.VMEM((KL, EMB, BLP), jnp.float32),
            pltpu.VMEM((KL, EMB, BLP), jnp.float32),
            pltpu.SemaphoreType.DMA,
            pltpu.SemaphoreType.DMA,
            pltpu.SemaphoreType.DMA,
            pltpu.SemaphoreType.DMA,
            pltpu.SemaphoreType.DMA,
            pltpu.SemaphoreType.DMA,
            pltpu.SemaphoreType.DMA,
            pltpu.SemaphoreType.DMA,
        ],
    )
    def gather(uids, iidsT, utab, itab, uout, iout,
               i0, i1, i2, i3, r0, r1, t0, t1,
               si0, si1, si2, si3, sg0, sg1, so0, so1):
        idx = (i0, i1, i2, i3)
        rows = (r0, r1)
        tb = (t0, t1)
        si = (si0, si1, si2, si3)
        sg = (sg0, sg1)
        so = (so0, so1)

        wid = lax.axis_index("s") * NC + lax.axis_index("c")
        bbase = wid * bpw

        def user_body(k, carry):
            ub0 = pl.multiple_of(bbase + k * BL, 8)
            pltpu.sync_copy(uids.at[pl.ds(ub0, BL)], i0.at[pl.ds(0, BL)])
            pltpu.async_copy(
                utab.at[i0.at[pl.ds(0, BL)]], r0.at[pl.ds(0, BL)], sg0
            ).wait()
            _transpose_block(r0, t0, 0, 0)
            pltpu.sync_copy(
                t0.at[0, :, pl.ds(0, BL)], uout.at[:, pl.ds(ub0, BL)])
            return carry

        lax.fori_loop(0, nblk, user_body, 0)

        def idx_starts(c, q):
            bi = lax.rem(c, nblk)
            l0 = lax.div(c, nblk) * KL
            b0 = bbase + bi * BL
            for lc in range(KL):
                pltpu.make_async_copy(
                    iidsT.at[l0 + lc, pl.ds(b0, BL)],
                    idx[q].at[pl.ds(lc * BL, BL)], si[q]).start()

        def idx_waits(c, q):
            bi = lax.rem(c, nblk)
            l0 = lax.div(c, nblk) * KL
            b0 = bbase + bi * BL
            for lc in range(KL):
                pltpu.make_async_copy(
                    iidsT.at[l0 + lc, pl.ds(b0, BL)],
                    idx[q].at[pl.ds(lc * BL, BL)], si[q]).wait()

        def store_op(c, p):
            bi = lax.rem(c, nblk)
            l0 = lax.div(c, nblk) * KL
            b0 = bbase + bi * BL
            return pltpu.make_async_copy(
                tb[p].at[:, :, pl.ds(0, BL)],
                iout.at[pl.ds(l0, KL), :, pl.ds(b0, BL)], so[p])

        for q in range(UNROLL):
            idx_starts(q, q)

        def outer(go, carry):
            for j in range(UNROLL):
                c = go * UNROLL + j
                @pl.when(c < n_chunks)
                def _():
                    idx_waits(c, j)
                    pltpu.make_async_copy(
                        itab.at[idx[j]], rows[j % 2], sg[j % 2]).start()
                cf = c - 1
                pj = (j + 1) % 2
                qf = (j + 3) % 4

                @pl.when((cf >= 0) & (cf < n_chunks))
                def _():
                    pltpu.make_async_copy(
                        itab.at[idx[qf]], rows[pj], sg[pj]).wait()
                    @pl.when(cf + UNROLL < n_chunks)
                    def _():
                        idx_starts(cf + UNROLL, qf)
                    @pl.when(cf >= 2)
                    def _():
                        store_op(cf - 2, pj).wait()
                    for lc in range(KL):
                        _transpose_block(rows[pj], tb[pj], lc, lc * BL)
                    store_op(cf, pj).start()
            return carry

        lax.fori_loop(0, n_chunks // UNROLL + 1, outer, 0)

        store_op(n_chunks - 2, (n_chunks - 2) % 2).wait()
        store_op(n_chunks - 1, (n_chunks - 1) % 2).wait()

    return gather


def kernel(user_ids, item_ids, user_table, item_table):
    b, l = item_ids.shape
    iT = jnp.transpose(item_ids).astype(jnp.int32)
    uids = user_ids.astype(jnp.int32)
    gather = _make_gather(uids.shape[0], b, l)
    uout_t, iout_t = gather(uids, iT, user_table, item_table)
    return (jnp.transpose(uout_t), jnp.transpose(iout_t, (2, 0, 1)))
